# Initial kernel scaffold; baseline (speedup 1.0000x reference)
import jax
import jax.numpy as jnp
from jax.experimental import pallas as pl


def kernel(x, adj_t, W1, b1, W2, b2):
    raise NotImplementedError("write your pallas kernel here")


# speedup vs baseline: 16.2334x; 16.2334x over previous
import functools

import jax
import jax.numpy as jnp
from jax import lax
from jax.experimental import pallas as pl
from jax.experimental.pallas import tpu as pltpu
from jax.experimental.pallas import tpu_sc as plsc

_NC = 2
_NS = 16
_NW = _NC * _NS
_LANES = 16
_CHUNK = 128

_SELU_SCALE = 1.0507009873554805
_SELU_ALPHA = 1.6732632423543772


def _sc_degree(dst):
    e = dst.shape[0]
    n = 10000
    nchunks = e // _CHUNK
    iters = (nchunks + _NW - 1) // _NW
    mesh = plsc.VectorSubcoreMesh(core_axis_name="c", subcore_axis_name="s")

    @functools.partial(
        pl.kernel,
        mesh=mesh,
        out_type=jax.ShapeDtypeStruct((_NW, n), jnp.float32),
        scratch_types=[
            pltpu.VMEM((n,), jnp.float32),
            pltpu.VMEM((_CHUNK,), jnp.int32),
        ],
        compiler_params=pltpu.CompilerParams(needs_layout_passes=False),
    )
    def k(dst_hbm, out_hbm, hist, idxb):
        c = lax.axis_index("c")
        s = lax.axis_index("s")
        wid = s * _NC + c

        def zbody(i, carry):
            hist[pl.ds(i * _LANES, _LANES)] = jnp.zeros((_LANES,), jnp.float32)
            return carry

        lax.fori_loop(0, n // _LANES, zbody, 0)

        ones16 = jnp.full((_LANES,), 1.0, jnp.float32)

        def body(j, carry):
            cid = wid + _NW * j

            @pl.when(cid < nchunks)
            def _():
                pltpu.sync_copy(dst_hbm.at[pl.ds(cid * _CHUNK, _CHUNK)], idxb)
                for t in range(_CHUNK // _LANES):
                    idx16 = idxb[pl.ds(t * _LANES, _LANES)]
                    plsc.addupdate_scatter(hist, [idx16], ones16)

            return carry

        lax.fori_loop(0, iters, body, 0)
        pltpu.sync_copy(hist, out_hbm.at[wid])

    return k(dst)


def _sc_aggregate(hs, src, dst, zeros_nd):
    n = hs.shape[0]
    e = src.shape[0]
    d = hs.shape[1]
    nchunks = e // _CHUNK
    iters = (nchunks + _NW - 1) // _NW
    rpt = (n // (8 * _NS)) * 8
    rlast = n - (_NS - 1) * rpt
    mesh = plsc.VectorSubcoreMesh(core_axis_name="c", subcore_axis_name="s")

    @functools.partial(
        pl.kernel,
        mesh=mesh,
        out_type=jax.ShapeDtypeStruct((_NC, n, d), jnp.float32),
        scratch_types=[
            pltpu.VMEM_SHARED((n, d), jnp.float32),
            pltpu.VMEM((_CHUNK,), jnp.int32),
            pltpu.VMEM((_CHUNK,), jnp.int32),
            pltpu.VMEM((_CHUNK, d), jnp.float32),
            pltpu.SemaphoreType.DMA,
        ],
    )
    def k(hs_hbm, src_hbm, dst_hbm, z_hbm, out_hbm, acc, idx_s, idx_d, rows, sem):
        c = lax.axis_index("c")
        s = lax.axis_index("s")
        wid = s * _NC + c
        r0 = pl.multiple_of(s * rpt, 8)

        @pl.when(s < _NS - 1)
        def _():
            pltpu.sync_copy(z_hbm.at[pl.ds(r0, rpt)], acc.at[pl.ds(r0, rpt)])

        @pl.when(s == _NS - 1)
        def _():
            pltpu.sync_copy(z_hbm.at[pl.ds(r0, rlast)], acc.at[pl.ds(r0, rlast)])

        plsc.subcore_barrier()

        def body(j, carry):
            cid = wid + _NW * j

            @pl.when(cid < nchunks)
            def _():
                base = cid * _CHUNK
                pltpu.sync_copy(src_hbm.at[pl.ds(base, _CHUNK)], idx_s)
                pltpu.sync_copy(dst_hbm.at[pl.ds(base, _CHUNK)], idx_d)
                pltpu.async_copy(hs_hbm.at[idx_s], rows, sem).wait()
                pltpu.sync_copy(rows, acc.at[idx_d], add=True)

            return carry

        lax.fori_loop(0, iters, body, 0)
        plsc.subcore_barrier()

        @pl.when(s < _NS - 1)
        def _():
            pltpu.sync_copy(acc.at[pl.ds(r0, rpt)], out_hbm.at[c, pl.ds(r0, rpt)])

        @pl.when(s == _NS - 1)
        def _():
            pltpu.sync_copy(acc.at[pl.ds(r0, rlast)], out_hbm.at[c, pl.ds(r0, rlast)])

    return k(hs, src, dst, zeros_nd)


def _tc_scale_matmul(x, w, dis):
    n, d = x.shape
    blk = 1000

    def body(x_ref, w_ref, dis_ref, o_ref):
        h = jnp.dot(x_ref[...], w_ref[...], preferred_element_type=jnp.float32)
        o_ref[...] = h * dis_ref[...]

    return pl.pallas_call(
        body,
        grid=(n // blk,),
        in_specs=[
            pl.BlockSpec((blk, d), lambda i: (i, 0)),
            pl.BlockSpec((d, d), lambda i: (0, 0)),
            pl.BlockSpec((blk, 1), lambda i: (i, 0)),
        ],
        out_specs=pl.BlockSpec((blk, d), lambda i: (i, 0)),
        out_shape=jax.ShapeDtypeStruct((n, d), jnp.float32),
    )(x, w, dis)


def _tc_combine_selu_matmul(y, hs, dis, b, w):
    n, d = hs.shape
    blk = 1000

    def body(y_ref, hs_ref, dis_ref, b_ref, w_ref, o_ref):
        t = dis_ref[...] * (y_ref[0] + y_ref[1] + hs_ref[...]) + b_ref[...]
        a = _SELU_SCALE * jnp.where(t > 0, t, _SELU_ALPHA * (jnp.exp(t) - 1.0))
        h = jnp.dot(a, w_ref[...], preferred_element_type=jnp.float32)
        o_ref[...] = h * dis_ref[...]

    return pl.pallas_call(
        body,
        grid=(n // blk,),
        in_specs=[
            pl.BlockSpec((_NC, blk, d), lambda i: (0, i, 0)),
            pl.BlockSpec((blk, d), lambda i: (i, 0)),
            pl.BlockSpec((blk, 1), lambda i: (i, 0)),
            pl.BlockSpec((1, d), lambda i: (0, 0)),
            pl.BlockSpec((d, d), lambda i: (0, 0)),
        ],
        out_specs=pl.BlockSpec((blk, d), lambda i: (i, 0)),
        out_shape=jax.ShapeDtypeStruct((n, d), jnp.float32),
    )(y, hs, dis, b, w)


def _tc_combine(y, hs, dis, b):
    n, d = hs.shape
    blk = 1000

    def body(y_ref, hs_ref, dis_ref, b_ref, o_ref):
        o_ref[...] = dis_ref[...] * (y_ref[0] + y_ref[1] + hs_ref[...]) + b_ref[...]

    return pl.pallas_call(
        body,
        grid=(n // blk,),
        in_specs=[
            pl.BlockSpec((_NC, blk, d), lambda i: (0, i, 0)),
            pl.BlockSpec((blk, d), lambda i: (i, 0)),
            pl.BlockSpec((blk, 1), lambda i: (i, 0)),
            pl.BlockSpec((1, d), lambda i: (0, 0)),
        ],
        out_specs=pl.BlockSpec((blk, d), lambda i: (i, 0)),
        out_shape=jax.ShapeDtypeStruct((n, d), jnp.float32),
    )(y, hs, dis, b)


def kernel(x, adj_t, W1, b1, W2, b2):
    n, d = x.shape
    src = adj_t[0]
    dst = adj_t[1]

    degp = _sc_degree(dst)
    deg = jnp.sum(degp, axis=0) + 1.0
    dis = lax.rsqrt(deg)[:, None]

    zeros_nd = jnp.zeros((n, d), jnp.float32)
    b1r = b1[None, :]
    b2r = b2[None, :]

    hs1 = _tc_scale_matmul(x, W1, dis)
    y1 = _sc_aggregate(hs1, src, dst, zeros_nd)
    hs2 = _tc_combine_selu_matmul(y1, hs1, dis, b1r, W2)
    y2 = _sc_aggregate(hs2, src, dst, zeros_nd)
    return _tc_combine(y2, hs2, dis, b2r)


# speedup vs baseline: 30.7579x; 1.8947x over previous
import functools

import jax
import jax.numpy as jnp
from jax import lax
from jax.experimental import pallas as pl
from jax.experimental.pallas import tpu as pltpu
from jax.experimental.pallas import tpu_sc as plsc

_NC = 2
_NS = 16
_NW = _NC * _NS
_LANES = 16
_CHUNK = 128
_NP = 10240

_SELU_SCALE = 1.0507009873554805
_SELU_ALPHA = 1.6732632423543772


def _sc_degree(dst2d, nchunks):
    maxc = dst2d.shape[0] // _NW
    mesh = plsc.VectorSubcoreMesh(core_axis_name="c", subcore_axis_name="s")

    @functools.partial(
        pl.kernel,
        mesh=mesh,
        out_type=jax.ShapeDtypeStruct((_NW, _NP), jnp.float32),
        scratch_types=[
            pltpu.VMEM((_NP,), jnp.float32),
            pltpu.VMEM((maxc, _CHUNK), jnp.int32),
        ],
        compiler_params=pltpu.CompilerParams(needs_layout_passes=False),
    )
    def k(dst_hbm, out_hbm, hist, idxb):
        c = lax.axis_index("c")
        s = lax.axis_index("s")
        wid = s * _NC + c
        c0 = pl.multiple_of(wid * maxc, 8)
        nch = jnp.minimum(jnp.maximum(nchunks - wid * maxc, 0), maxc)

        pltpu.sync_copy(dst_hbm.at[pl.ds(c0, maxc)], idxb)

        def zbody(i, carry):
            hist[pl.ds(i * _LANES, _LANES)] = jnp.zeros((_LANES,), jnp.float32)
            return carry

        lax.fori_loop(0, _NP // _LANES, zbody, 0)

        ones16 = jnp.full((_LANES,), 1.0, jnp.float32)

        def body(j, carry):
            @pl.when(j < nch)
            def _():
                for t in range(_CHUNK // _LANES):
                    idx16 = idxb[j, pl.ds(t * _LANES, _LANES)]
                    plsc.addupdate_scatter(hist, [idx16], ones16)

            return carry

        lax.fori_loop(0, maxc, body, 0)
        pltpu.sync_copy(hist, out_hbm.at[wid])

    return k(dst2d)


_W = 16


def _sc_aggregate(hs, src2d, dst2d, nchunks, zeros_nd):
    d = hs.shape[1]
    maxc = src2d.shape[0] // _NW
    rpt = _NP // _NS
    mesh = plsc.VectorSubcoreMesh(core_axis_name="c", subcore_axis_name="s")

    @functools.partial(
        pl.kernel,
        mesh=mesh,
        out_type=jax.ShapeDtypeStruct((_NC, _NP, d), jnp.float32),
        scratch_types=[
            pltpu.VMEM_SHARED((_NP, d), jnp.float32),
            pltpu.VMEM((_W, _CHUNK), jnp.int32),
            pltpu.VMEM((_W, _CHUNK), jnp.int32),
            pltpu.VMEM((_CHUNK, d), jnp.float32),
            pltpu.VMEM((_CHUNK, d), jnp.float32),
            pltpu.SemaphoreType.DMA,
            pltpu.SemaphoreType.DMA,
        ],
    )
    def k(hs_hbm, src_hbm, dst_hbm, z_hbm, out_hbm,
          acc, idx_s, idx_d, rows0, rows1, sem0, sem1):
        c = lax.axis_index("c")
        s = lax.axis_index("s")
        wid = s * _NC + c
        c0 = pl.multiple_of(wid * maxc, 8)
        nch = jnp.minimum(jnp.maximum(nchunks - wid * maxc, 0), maxc)
        r0 = pl.multiple_of(s * rpt, 8)

        pltpu.sync_copy(z_hbm.at[pl.ds(r0, rpt)], acc.at[pl.ds(r0, rpt)])
        plsc.subcore_barrier()

        bufs = ((rows0, sem0), (rows1, sem1))

        def win_body(w, carry):
            wbase = pl.multiple_of(c0 + w * _W, 8)
            pltpu.sync_copy(src_hbm.at[pl.ds(wbase, _W)], idx_s)
            pltpu.sync_copy(dst_hbm.at[pl.ds(wbase, _W)], idx_d)
            j0 = w * _W

            @pl.when(j0 < nch)
            def _():
                pltpu.make_async_copy(hs_hbm.at[idx_s.at[0]], rows0, sem0).start()

            def pair_body(t2, carry2):
                for b in range(2):
                    t = t2 * 2 + b
                    j = j0 + t
                    rb, sb = bufs[b]
                    rn, sn = bufs[1 - b]

                    @pl.when(jnp.logical_and(t + 1 < _W, j + 1 < nch))
                    def _():
                        pltpu.make_async_copy(
                            hs_hbm.at[idx_s.at[t + 1]], rn, sn).start()

                    @pl.when(j < nch)
                    def _():
                        pltpu.make_async_copy(
                            hs_hbm.at[idx_s.at[t]], rb, sb).wait()
                        pltpu.sync_copy(rb, acc.at[idx_d.at[t]], add=True)

                return carry2

            lax.fori_loop(0, _W // 2, pair_body, 0)
            return carry

        lax.fori_loop(0, maxc // _W, win_body, 0)
        plsc.subcore_barrier()
        pltpu.sync_copy(acc.at[pl.ds(r0, rpt)], out_hbm.at[c, pl.ds(r0, rpt)])

    return k(hs, src2d, dst2d, zeros_nd)


def _tc_scale_matmul(x, w, dis):
    n, d = x.shape
    blk = 1000

    def body(x_ref, w_ref, dis_ref, o_ref):
        h = jnp.dot(x_ref[...], w_ref[...], preferred_element_type=jnp.float32)
        o_ref[...] = h * dis_ref[...]

    return pl.pallas_call(
        body,
        grid=(n // blk,),
        in_specs=[
            pl.BlockSpec((blk, d), lambda i: (i, 0)),
            pl.BlockSpec((d, d), lambda i: (0, 0)),
            pl.BlockSpec((blk, 1), lambda i: (i, 0)),
        ],
        out_specs=pl.BlockSpec((blk, d), lambda i: (i, 0)),
        out_shape=jax.ShapeDtypeStruct((n, d), jnp.float32),
    )(x, w, dis)


def _tc_combine_selu_matmul(y, hs, dis, b, w):
    n, d = hs.shape
    blk = 1000

    def body(y_ref, hs_ref, dis_ref, b_ref, w_ref, o_ref):
        t = dis_ref[...] * (y_ref[0] + y_ref[1] + hs_ref[...]) + b_ref[...]
        a = _SELU_SCALE * jnp.where(t > 0, t, _SELU_ALPHA * (jnp.exp(t) - 1.0))
        h = jnp.dot(a, w_ref[...], preferred_element_type=jnp.float32)
        o_ref[...] = h * dis_ref[...]

    return pl.pallas_call(
        body,
        grid=(n // blk,),
        in_specs=[
            pl.BlockSpec((_NC, blk, d), lambda i: (0, i, 0)),
            pl.BlockSpec((blk, d), lambda i: (i, 0)),
            pl.BlockSpec((blk, 1), lambda i: (i, 0)),
            pl.BlockSpec((1, d), lambda i: (0, 0)),
            pl.BlockSpec((d, d), lambda i: (0, 0)),
        ],
        out_specs=pl.BlockSpec((blk, d), lambda i: (i, 0)),
        out_shape=jax.ShapeDtypeStruct((n, d), jnp.float32),
    )(y, hs, dis, b, w)


def _tc_combine(y, hs, dis, b):
    n, d = hs.shape
    blk = 1000

    def body(y_ref, hs_ref, dis_ref, b_ref, o_ref):
        o_ref[...] = dis_ref[...] * (y_ref[0] + y_ref[1] + hs_ref[...]) + b_ref[...]

    return pl.pallas_call(
        body,
        grid=(n // blk,),
        in_specs=[
            pl.BlockSpec((_NC, blk, d), lambda i: (0, i, 0)),
            pl.BlockSpec((blk, d), lambda i: (i, 0)),
            pl.BlockSpec((blk, 1), lambda i: (i, 0)),
            pl.BlockSpec((1, d), lambda i: (0, 0)),
        ],
        out_specs=pl.BlockSpec((blk, d), lambda i: (i, 0)),
        out_shape=jax.ShapeDtypeStruct((n, d), jnp.float32),
    )(y, hs, dis, b)


def kernel(x, adj_t, W1, b1, W2, b2):
    n, d = x.shape
    e = adj_t.shape[1]
    nchunks = e // _CHUNK
    maxc = -(-nchunks // _NW)
    maxc = -(-maxc // _W) * _W
    pad = _NW * maxc * _CHUNK - e

    src2d = jnp.concatenate(
        [adj_t[0], jnp.zeros((pad,), adj_t.dtype)]).reshape(_NW * maxc, _CHUNK)
    dst2d = jnp.concatenate(
        [adj_t[1], jnp.zeros((pad,), adj_t.dtype)]).reshape(_NW * maxc, _CHUNK)

    degp = _sc_degree(dst2d, nchunks)
    deg = jnp.sum(degp, axis=0) + 1.0
    dis = lax.rsqrt(deg)[:, None]

    zeros_nd = jnp.zeros((_NP, d), jnp.float32)
    b1r = b1[None, :]
    b2r = b2[None, :]

    hs1 = _tc_scale_matmul(x, W1, dis[:n])
    y1 = _sc_aggregate(hs1, src2d, dst2d, nchunks, zeros_nd)
    hs2 = _tc_combine_selu_matmul(y1, hs1, dis[:n], b1r, W2)
    y2 = _sc_aggregate(hs2, src2d, dst2d, nchunks, zeros_nd)
    return _tc_combine(y2, hs2, dis[:n], b2r)


# speedup vs baseline: 31.2435x; 1.0158x over previous
import functools

import jax
import jax.numpy as jnp
from jax import lax
from jax.experimental import pallas as pl
from jax.experimental.pallas import tpu as pltpu
from jax.experimental.pallas import tpu_sc as plsc

_NC = 2
_NS = 16
_NW = _NC * _NS
_LANES = 16
_CHUNK = 128
_NP = 10240

_SELU_SCALE = 1.0507009873554805
_SELU_ALPHA = 1.6732632423543772


def _sc_degree(dst2d, nchunks):
    maxc = dst2d.shape[0] // _NW
    mesh = plsc.VectorSubcoreMesh(core_axis_name="c", subcore_axis_name="s")

    @functools.partial(
        pl.kernel,
        mesh=mesh,
        out_type=jax.ShapeDtypeStruct((_NW, _NP), jnp.float32),
        scratch_types=[
            pltpu.VMEM((_NP,), jnp.float32),
            pltpu.VMEM((maxc, _CHUNK), jnp.int32),
        ],
        compiler_params=pltpu.CompilerParams(needs_layout_passes=False),
    )
    def k(dst_hbm, out_hbm, hist, idxb):
        c = lax.axis_index("c")
        s = lax.axis_index("s")
        wid = s * _NC + c
        c0 = pl.multiple_of(wid * maxc, 8)
        nch = jnp.minimum(jnp.maximum(nchunks - wid * maxc, 0), maxc)

        pltpu.sync_copy(dst_hbm.at[pl.ds(c0, maxc)], idxb)

        def zbody(i, carry):
            hist[pl.ds(i * _LANES, _LANES)] = jnp.zeros((_LANES,), jnp.float32)
            return carry

        lax.fori_loop(0, _NP // _LANES, zbody, 0)

        ones16 = jnp.full((_LANES,), 1.0, jnp.float32)

        def body(j, carry):
            @pl.when(j < nch)
            def _():
                for t in range(_CHUNK // _LANES):
                    idx16 = idxb[j, pl.ds(t * _LANES, _LANES)]
                    plsc.addupdate_scatter(hist, [idx16], ones16)

            return carry

        lax.fori_loop(0, maxc, body, 0)
        pltpu.sync_copy(hist, out_hbm.at[wid])

    return k(dst2d)


_W = 16


def _sc_aggregate(hs, src2d, dst2d, nchunks):
    n, d = hs.shape
    maxc = src2d.shape[0] // _NW
    rpt = _NP // _NS
    rlast = n - (_NS - 1) * rpt
    mesh = plsc.VectorSubcoreMesh(core_axis_name="c", subcore_axis_name="s")

    @functools.partial(
        pl.kernel,
        mesh=mesh,
        out_type=jax.ShapeDtypeStruct((_NC, _NP, d), jnp.float32),
        scratch_types=[
            pltpu.VMEM_SHARED((_NP, d), jnp.float32),
            pltpu.VMEM((_W, _CHUNK), jnp.int32),
            pltpu.VMEM((_W, _CHUNK), jnp.int32),
            pltpu.VMEM((_CHUNK, d), jnp.float32),
            pltpu.VMEM((_CHUNK, d), jnp.float32),
            pltpu.SemaphoreType.DMA,
            pltpu.SemaphoreType.DMA,
        ],
    )
    def k(hs_hbm, src_hbm, dst_hbm, out_hbm,
          acc, idx_s, idx_d, rows0, rows1, sem0, sem1):
        c = lax.axis_index("c")
        s = lax.axis_index("s")
        wid = s * _NC + c
        c0 = pl.multiple_of(wid * maxc, 8)
        nch = jnp.minimum(jnp.maximum(nchunks - wid * maxc, 0), maxc)
        r0 = pl.multiple_of(s * rpt, 8)

        @pl.when(jnp.logical_and(c == 0, s < _NS - 1))
        def _():
            pltpu.sync_copy(hs_hbm.at[pl.ds(r0, rpt)], acc.at[pl.ds(r0, rpt)])

        @pl.when(jnp.logical_and(c == 0, s == _NS - 1))
        def _():
            pltpu.sync_copy(hs_hbm.at[pl.ds(r0, rlast)], acc.at[pl.ds(r0, rlast)])

        @pl.when(c == 1)
        def _():
            z16 = jnp.zeros((_LANES,), jnp.float32)

            def zb(r, carry):
                for t in range(d // _LANES):
                    rows0[r, pl.ds(t * _LANES, _LANES)] = z16
                return carry

            lax.fori_loop(0, _CHUNK, zb, 0)
            for p in range(rpt // _CHUNK):
                pltpu.sync_copy(
                    rows0, acc.at[pl.ds(pl.multiple_of(r0 + p * _CHUNK, 8),
                                        _CHUNK)])

        plsc.subcore_barrier()

        bufs = ((rows0, sem0), (rows1, sem1))

        def win_body(w, carry):
            wbase = pl.multiple_of(c0 + w * _W, 8)
            pltpu.sync_copy(src_hbm.at[pl.ds(wbase, _W)], idx_s)
            pltpu.sync_copy(dst_hbm.at[pl.ds(wbase, _W)], idx_d)
            j0 = w * _W

            @pl.when(j0 < nch)
            def _():
                pltpu.make_async_copy(hs_hbm.at[idx_s.at[0]], rows0, sem0).start()

            def pair_body(t2, carry2):
                for b in range(2):
                    t = t2 * 2 + b
                    j = j0 + t
                    rb, sb = bufs[b]
                    rn, sn = bufs[1 - b]

                    @pl.when(jnp.logical_and(t + 1 < _W, j + 1 < nch))
                    def _():
                        pltpu.make_async_copy(
                            hs_hbm.at[idx_s.at[t + 1]], rn, sn).start()

                    @pl.when(j < nch)
                    def _():
                        pltpu.make_async_copy(
                            hs_hbm.at[idx_s.at[t]], rb, sb).wait()
                        pltpu.sync_copy(rb, acc.at[idx_d.at[t]], add=True)

                return carry2

            lax.fori_loop(0, _W // 2, pair_body, 0)
            return carry

        lax.fori_loop(0, maxc // _W, win_body, 0)
        plsc.subcore_barrier()
        pltpu.sync_copy(acc.at[pl.ds(r0, rpt)], out_hbm.at[c, pl.ds(r0, rpt)])

    return k(hs, src2d, dst2d)


def _tc_scale_matmul(x, w, dis):
    n, d = x.shape
    blk = 1000

    def body(x_ref, w_ref, dis_ref, o_ref):
        h = jnp.dot(x_ref[...], w_ref[...], preferred_element_type=jnp.float32)
        o_ref[...] = h * dis_ref[...]

    return pl.pallas_call(
        body,
        grid=(n // blk,),
        in_specs=[
            pl.BlockSpec((blk, d), lambda i: (i, 0)),
            pl.BlockSpec((d, d), lambda i: (0, 0)),
            pl.BlockSpec((blk, 1), lambda i: (i, 0)),
        ],
        out_specs=pl.BlockSpec((blk, d), lambda i: (i, 0)),
        out_shape=jax.ShapeDtypeStruct((n, d), jnp.float32),
    )(x, w, dis)


def _tc_combine_selu_matmul(y, n, dis, b, w):
    d = y.shape[2]
    blk = 1000

    def body(y_ref, dis_ref, b_ref, w_ref, o_ref):
        t = dis_ref[...] * (y_ref[0] + y_ref[1]) + b_ref[...]
        a = _SELU_SCALE * jnp.where(t > 0, t, _SELU_ALPHA * (jnp.exp(t) - 1.0))
        h = jnp.dot(a, w_ref[...], preferred_element_type=jnp.float32)
        o_ref[...] = h * dis_ref[...]

    return pl.pallas_call(
        body,
        grid=(n // blk,),
        in_specs=[
            pl.BlockSpec((_NC, blk, d), lambda i: (0, i, 0)),
            pl.BlockSpec((blk, 1), lambda i: (i, 0)),
            pl.BlockSpec((1, d), lambda i: (0, 0)),
            pl.BlockSpec((d, d), lambda i: (0, 0)),
        ],
        out_specs=pl.BlockSpec((blk, d), lambda i: (i, 0)),
        out_shape=jax.ShapeDtypeStruct((n, d), jnp.float32),
    )(y, dis, b, w)


def _tc_combine(y, n, dis, b):
    d = y.shape[2]
    blk = 1000

    def body(y_ref, dis_ref, b_ref, o_ref):
        o_ref[...] = dis_ref[...] * (y_ref[0] + y_ref[1]) + b_ref[...]

    return pl.pallas_call(
        body,
        grid=(n // blk,),
        in_specs=[
            pl.BlockSpec((_NC, blk, d), lambda i: (0, i, 0)),
            pl.BlockSpec((blk, 1), lambda i: (i, 0)),
            pl.BlockSpec((1, d), lambda i: (0, 0)),
        ],
        out_specs=pl.BlockSpec((blk, d), lambda i: (i, 0)),
        out_shape=jax.ShapeDtypeStruct((n, d), jnp.float32),
    )(y, dis, b)


def kernel(x, adj_t, W1, b1, W2, b2):
    n, d = x.shape
    e = adj_t.shape[1]
    nchunks = e // _CHUNK
    maxc = -(-nchunks // _NW)
    maxc = -(-maxc // _W) * _W
    pad = _NW * maxc * _CHUNK - e

    src2d = jnp.concatenate(
        [adj_t[0], jnp.zeros((pad,), adj_t.dtype)]).reshape(_NW * maxc, _CHUNK)
    dst2d = jnp.concatenate(
        [adj_t[1], jnp.zeros((pad,), adj_t.dtype)]).reshape(_NW * maxc, _CHUNK)

    degp = _sc_degree(dst2d, nchunks)
    deg = jnp.sum(degp, axis=0) + 1.0
    dis = lax.rsqrt(deg)[:, None]

    b1r = b1[None, :]
    b2r = b2[None, :]

    hs1 = _tc_scale_matmul(x, W1, dis[:n])
    y1 = _sc_aggregate(hs1, src2d, dst2d, nchunks)
    hs2 = _tc_combine_selu_matmul(y1, n, dis[:n], b1r, W2)
    y2 = _sc_aggregate(hs2, src2d, dst2d, nchunks)
    return _tc_combine(y2, n, dis[:n], b2r)


# speedup vs baseline: 33.4230x; 1.0698x over previous
import functools

import jax
import jax.numpy as jnp
from jax import lax
from jax.experimental import pallas as pl
from jax.experimental.pallas import tpu as pltpu
from jax.experimental.pallas import tpu_sc as plsc

_NC = 2
_NS = 16
_NW = _NC * _NS
_LANES = 16
_CHUNK = 128
_NP = 10240

_SELU_SCALE = 1.0507009873554805
_SELU_ALPHA = 1.6732632423543772


def _sc_degree(dst2d, nchunks):
    maxc = dst2d.shape[0] // _NW
    mesh = plsc.VectorSubcoreMesh(core_axis_name="c", subcore_axis_name="s")

    @functools.partial(
        pl.kernel,
        mesh=mesh,
        out_type=jax.ShapeDtypeStruct((_NW, _NP), jnp.float32),
        scratch_types=[
            pltpu.VMEM((_NP,), jnp.float32),
            pltpu.VMEM((maxc, _CHUNK), jnp.int32),
        ],
        compiler_params=pltpu.CompilerParams(needs_layout_passes=False),
    )
    def k(dst_hbm, out_hbm, hist, idxb):
        c = lax.axis_index("c")
        s = lax.axis_index("s")
        wid = s * _NC + c
        c0 = pl.multiple_of(wid * maxc, 8)
        nch = jnp.minimum(jnp.maximum(nchunks - wid * maxc, 0), maxc)

        pltpu.sync_copy(dst_hbm.at[pl.ds(c0, maxc)], idxb)

        def zbody(i, carry):
            hist[pl.ds(i * _LANES, _LANES)] = jnp.zeros((_LANES,), jnp.float32)
            return carry

        lax.fori_loop(0, _NP // _LANES, zbody, 0)

        ones16 = jnp.full((_LANES,), 1.0, jnp.float32)

        def body(j, carry):
            @pl.when(j < nch)
            def _():
                for t in range(_CHUNK // _LANES):
                    idx16 = idxb[j, pl.ds(t * _LANES, _LANES)]
                    plsc.addupdate_scatter(hist, [idx16], ones16)

            return carry

        lax.fori_loop(0, maxc, body, 0)
        pltpu.sync_copy(hist, out_hbm.at[wid])

    return k(dst2d)


_W = 16


def _sc_aggregate(hs, src2d, dst2d, nchunks):
    n, d = hs.shape
    maxc = src2d.shape[0] // _NW
    rpt = _NP // _NS
    rlast = n - (_NS - 1) * rpt
    mesh = plsc.VectorSubcoreMesh(core_axis_name="c", subcore_axis_name="s")

    @functools.partial(
        pl.kernel,
        mesh=mesh,
        out_type=jax.ShapeDtypeStruct((_NC, _NP, d), jnp.float32),
        scratch_types=[
            pltpu.VMEM_SHARED((_NP, d), jnp.float32),
            pltpu.VMEM((2 * _W, _CHUNK), jnp.int32),
            pltpu.VMEM((2 * _W, _CHUNK), jnp.int32),
            pltpu.VMEM((_CHUNK, d), jnp.float32),
            pltpu.VMEM((_CHUNK, d), jnp.float32),
            pltpu.SemaphoreType.DMA,
            pltpu.SemaphoreType.DMA,
            pltpu.SemaphoreType.DMA,
            pltpu.SemaphoreType.DMA,
        ],
    )
    def k(hs_hbm, src_hbm, dst_hbm, out_hbm,
          acc, idx_s, idx_d, rows0, rows1, sem0, sem1, semis, semid):
        c = lax.axis_index("c")
        s = lax.axis_index("s")
        wid = s * _NC + c
        c0 = pl.multiple_of(wid * maxc, 8)
        nch = jnp.minimum(jnp.maximum(nchunks - wid * maxc, 0), maxc)
        r0 = pl.multiple_of(s * rpt, 8)

        @pl.when(jnp.logical_and(c == 0, s < _NS - 1))
        def _():
            pltpu.sync_copy(hs_hbm.at[pl.ds(r0, rpt)], acc.at[pl.ds(r0, rpt)])

        @pl.when(jnp.logical_and(c == 0, s == _NS - 1))
        def _():
            pltpu.sync_copy(hs_hbm.at[pl.ds(r0, rlast)], acc.at[pl.ds(r0, rlast)])

        @pl.when(c == 1)
        def _():
            z16 = jnp.zeros((_LANES,), jnp.float32)

            def zb(r, carry):
                for t in range(d // _LANES):
                    rows0[r, pl.ds(t * _LANES, _LANES)] = z16
                return carry

            lax.fori_loop(0, _CHUNK, zb, 0)
            for p in range(rpt // _CHUNK):
                pltpu.sync_copy(
                    rows0, acc.at[pl.ds(pl.multiple_of(r0 + p * _CHUNK, 8),
                                        _CHUNK)])

        plsc.subcore_barrier()

        bufs = ((rows0, sem0), (rows1, sem1))
        nwin = maxc // _W

        pltpu.sync_copy(src_hbm.at[pl.ds(c0, _W)], idx_s.at[pl.ds(0, _W)])
        pltpu.sync_copy(dst_hbm.at[pl.ds(c0, _W)], idx_d.at[pl.ds(0, _W)])

        @pl.when(0 < nch)
        def _():
            pltpu.make_async_copy(hs_hbm.at[idx_s.at[0]], rows0, sem0).start()

        def win_body(w, carry):
            par = w % 2
            off = par * _W
            offn = _W - off
            j0 = w * _W
            nb = pl.multiple_of(c0 + (w + 1) * _W, 8)

            @pl.when(w + 1 < nwin)
            def _():
                pltpu.make_async_copy(
                    src_hbm.at[pl.ds(nb, _W)],
                    idx_s.at[pl.ds(offn, _W)], semis).start()
                pltpu.make_async_copy(
                    dst_hbm.at[pl.ds(nb, _W)],
                    idx_d.at[pl.ds(offn, _W)], semid).start()

            def pair_body(t2, carry2):
                @pl.when(jnp.logical_and(t2 == 3, w + 1 < nwin))
                def _():
                    pltpu.make_async_copy(
                        src_hbm.at[pl.ds(nb, _W)],
                        idx_s.at[pl.ds(offn, _W)], semis).wait()
                    pltpu.make_async_copy(
                        dst_hbm.at[pl.ds(nb, _W)],
                        idx_d.at[pl.ds(offn, _W)], semid).wait()

                for b in range(2):
                    t = t2 * 2 + b
                    j = j0 + t
                    rb, sb = bufs[b]
                    rn, sn = bufs[1 - b]
                    nxt_row = jnp.where(t + 1 < _W, off + t + 1, offn)

                    @pl.when(j + 1 < nch)
                    def _():
                        pltpu.make_async_copy(
                            hs_hbm.at[idx_s.at[nxt_row]], rn, sn).start()

                    @pl.when(j < nch)
                    def _():
                        pltpu.make_async_copy(
                            hs_hbm.at[idx_s.at[off + t]], rb, sb).wait()
                        pltpu.sync_copy(rb, acc.at[idx_d.at[off + t]], add=True)

                return carry2

            lax.fori_loop(0, _W // 2, pair_body, 0)
            return carry

        lax.fori_loop(0, nwin, win_body, 0)
        plsc.subcore_barrier()
        pltpu.sync_copy(acc.at[pl.ds(r0, rpt)], out_hbm.at[c, pl.ds(r0, rpt)])

    return k(hs, src2d, dst2d)


def _tc_scale_matmul(x, w, dis):
    n, d = x.shape
    blk = 1000

    def body(x_ref, w_ref, dis_ref, o_ref):
        h = jnp.dot(x_ref[...], w_ref[...], preferred_element_type=jnp.float32)
        o_ref[...] = h * dis_ref[...]

    return pl.pallas_call(
        body,
        grid=(n // blk,),
        in_specs=[
            pl.BlockSpec((blk, d), lambda i: (i, 0)),
            pl.BlockSpec((d, d), lambda i: (0, 0)),
            pl.BlockSpec((blk, 1), lambda i: (i, 0)),
        ],
        out_specs=pl.BlockSpec((blk, d), lambda i: (i, 0)),
        out_shape=jax.ShapeDtypeStruct((n, d), jnp.float32),
    )(x, w, dis)


def _tc_combine_selu_matmul(y, n, dis, b, w):
    d = y.shape[2]
    blk = 1000

    def body(y_ref, dis_ref, b_ref, w_ref, o_ref):
        t = dis_ref[...] * (y_ref[0] + y_ref[1]) + b_ref[...]
        a = _SELU_SCALE * jnp.where(t > 0, t, _SELU_ALPHA * (jnp.exp(t) - 1.0))
        h = jnp.dot(a, w_ref[...], preferred_element_type=jnp.float32)
        o_ref[...] = h * dis_ref[...]

    return pl.pallas_call(
        body,
        grid=(n // blk,),
        in_specs=[
            pl.BlockSpec((_NC, blk, d), lambda i: (0, i, 0)),
            pl.BlockSpec((blk, 1), lambda i: (i, 0)),
            pl.BlockSpec((1, d), lambda i: (0, 0)),
            pl.BlockSpec((d, d), lambda i: (0, 0)),
        ],
        out_specs=pl.BlockSpec((blk, d), lambda i: (i, 0)),
        out_shape=jax.ShapeDtypeStruct((n, d), jnp.float32),
    )(y, dis, b, w)


def _tc_combine(y, n, dis, b):
    d = y.shape[2]
    blk = 1000

    def body(y_ref, dis_ref, b_ref, o_ref):
        o_ref[...] = dis_ref[...] * (y_ref[0] + y_ref[1]) + b_ref[...]

    return pl.pallas_call(
        body,
        grid=(n // blk,),
        in_specs=[
            pl.BlockSpec((_NC, blk, d), lambda i: (0, i, 0)),
            pl.BlockSpec((blk, 1), lambda i: (i, 0)),
            pl.BlockSpec((1, d), lambda i: (0, 0)),
        ],
        out_specs=pl.BlockSpec((blk, d), lambda i: (i, 0)),
        out_shape=jax.ShapeDtypeStruct((n, d), jnp.float32),
    )(y, dis, b)


def kernel(x, adj_t, W1, b1, W2, b2):
    n, d = x.shape
    e = adj_t.shape[1]
    nchunks = e // _CHUNK
    maxc = -(-nchunks // _NW)
    maxc = -(-maxc // _W) * _W
    pad = _NW * maxc * _CHUNK - e

    src2d = jnp.concatenate(
        [adj_t[0], jnp.zeros((pad,), adj_t.dtype)]).reshape(_NW * maxc, _CHUNK)
    dst2d = jnp.concatenate(
        [adj_t[1], jnp.zeros((pad,), adj_t.dtype)]).reshape(_NW * maxc, _CHUNK)

    degp = _sc_degree(dst2d, nchunks)
    deg = jnp.sum(degp, axis=0) + 1.0
    dis = lax.rsqrt(deg)[:, None]

    b1r = b1[None, :]
    b2r = b2[None, :]

    hs1 = _tc_scale_matmul(x, W1, dis[:n])
    y1 = _sc_aggregate(hs1, src2d, dst2d, nchunks)
    hs2 = _tc_combine_selu_matmul(y1, n, dis[:n], b1r, W2)
    y2 = _sc_aggregate(hs2, src2d, dst2d, nchunks)
    return _tc_combine(y2, n, dis[:n], b2r)


# speedup vs baseline: 33.9833x; 1.0168x over previous
import functools

import jax
import jax.numpy as jnp
from jax import lax
from jax.experimental import pallas as pl
from jax.experimental.pallas import tpu as pltpu
from jax.experimental.pallas import tpu_sc as plsc

_NC = 2
_NS = 16
_NW = _NC * _NS
_LANES = 16
_CHUNK = 128
_NP = 10240

_SELU_SCALE = 1.0507009873554805
_SELU_ALPHA = 1.6732632423543772


def _sc_degree(dst2d, nchunks):
    maxc = dst2d.shape[0] // _NW
    mesh = plsc.VectorSubcoreMesh(core_axis_name="c", subcore_axis_name="s")

    @functools.partial(
        pl.kernel,
        mesh=mesh,
        out_type=jax.ShapeDtypeStruct((_NW, _NP), jnp.float32),
        scratch_types=[
            pltpu.VMEM((_NP,), jnp.float32),
            pltpu.VMEM((maxc, _CHUNK), jnp.int32),
        ],
        compiler_params=pltpu.CompilerParams(needs_layout_passes=False),
    )
    def k(dst_hbm, out_hbm, hist, idxb):
        c = lax.axis_index("c")
        s = lax.axis_index("s")
        wid = s * _NC + c
        c0 = pl.multiple_of(wid * maxc, 8)
        nch = jnp.minimum(jnp.maximum(nchunks - wid * maxc, 0), maxc)

        pltpu.sync_copy(dst_hbm.at[pl.ds(c0, maxc)], idxb)

        def zbody(i, carry):
            hist[pl.ds(i * _LANES, _LANES)] = jnp.zeros((_LANES,), jnp.float32)
            return carry

        lax.fori_loop(0, _NP // _LANES, zbody, 0)

        ones16 = jnp.full((_LANES,), 1.0, jnp.float32)

        def body(j, carry):
            @pl.when(j < nch)
            def _():
                for t in range(_CHUNK // _LANES):
                    idx16 = idxb[j, pl.ds(t * _LANES, _LANES)]
                    plsc.addupdate_scatter(hist, [idx16], ones16)

            return carry

        lax.fori_loop(0, maxc, body, 0)
        pltpu.sync_copy(hist, out_hbm.at[wid])

    return k(dst2d)


_W = 16


def _sc_aggregate(hs, src2d, dst2d, nchunks):
    n, d = hs.shape
    maxc = src2d.shape[0] // _NW
    rpt = _NP // _NS
    rlast = n - (_NS - 1) * rpt
    mesh = plsc.VectorSubcoreMesh(core_axis_name="c", subcore_axis_name="s")

    @functools.partial(
        pl.kernel,
        mesh=mesh,
        out_type=jax.ShapeDtypeStruct((_NC, _NP, d), jnp.float32),
        scratch_types=[
            pltpu.VMEM_SHARED((_NP, d), jnp.float32),
            pltpu.VMEM((2 * _W, _CHUNK), jnp.int32),
            pltpu.VMEM((2 * _W, _CHUNK), jnp.int32),
            pltpu.VMEM((_CHUNK, d), jnp.float32),
            pltpu.VMEM((_CHUNK, d), jnp.float32),
            pltpu.SemaphoreType.DMA,
            pltpu.SemaphoreType.DMA,
            pltpu.SemaphoreType.DMA,
            pltpu.SemaphoreType.DMA,
        ],
    )
    def k(hs_hbm, src_hbm, dst_hbm, out_hbm,
          acc, idx_s, idx_d, rows0, rows1, sem0, sem1, semis, semid):
        c = lax.axis_index("c")
        s = lax.axis_index("s")
        wid = s * _NC + c
        c0 = pl.multiple_of(wid * maxc, 8)
        nch = jnp.minimum(jnp.maximum(nchunks - wid * maxc, 0), maxc)
        r0 = pl.multiple_of(s * rpt, 8)

        @pl.when(jnp.logical_and(c == 0, s < _NS - 1))
        def _():
            pltpu.sync_copy(hs_hbm.at[pl.ds(r0, rpt)], acc.at[pl.ds(r0, rpt)])

        @pl.when(jnp.logical_and(c == 0, s == _NS - 1))
        def _():
            pltpu.sync_copy(hs_hbm.at[pl.ds(r0, rlast)], acc.at[pl.ds(r0, rlast)])

        @pl.when(c == 1)
        def _():
            z16 = jnp.zeros((_LANES,), jnp.float32)

            def zb(r, carry):
                for t in range(d // _LANES):
                    rows0[r, pl.ds(t * _LANES, _LANES)] = z16
                return carry

            lax.fori_loop(0, _CHUNK, zb, 0)
            for p in range(rpt // _CHUNK):
                pltpu.sync_copy(
                    rows0, acc.at[pl.ds(pl.multiple_of(r0 + p * _CHUNK, 8),
                                        _CHUNK)])

        plsc.subcore_barrier()

        bufs = ((rows0, sem0), (rows1, sem1))
        nwin = maxc // _W

        pltpu.sync_copy(src_hbm.at[pl.ds(c0, _W)], idx_s.at[pl.ds(0, _W)])
        pltpu.sync_copy(dst_hbm.at[pl.ds(c0, _W)], idx_d.at[pl.ds(0, _W)])

        @pl.when(0 < nch)
        def _():
            pltpu.make_async_copy(hs_hbm.at[idx_s.at[0]], rows0, sem0).start()

        def win_body(w, carry):
            par = w % 2
            off = par * _W
            offn = _W - off
            j0 = w * _W
            nb = pl.multiple_of(c0 + (w + 1) * _W, 8)

            @pl.when(w + 1 < nwin)
            def _():
                pltpu.make_async_copy(
                    src_hbm.at[pl.ds(nb, _W)],
                    idx_s.at[pl.ds(offn, _W)], semis).start()
                pltpu.make_async_copy(
                    dst_hbm.at[pl.ds(nb, _W)],
                    idx_d.at[pl.ds(offn, _W)], semid).start()

            def pair_body(t2, carry2):
                @pl.when(jnp.logical_and(t2 == 3, w + 1 < nwin))
                def _():
                    pltpu.make_async_copy(
                        src_hbm.at[pl.ds(nb, _W)],
                        idx_s.at[pl.ds(offn, _W)], semis).wait()
                    pltpu.make_async_copy(
                        dst_hbm.at[pl.ds(nb, _W)],
                        idx_d.at[pl.ds(offn, _W)], semid).wait()

                for b in range(2):
                    t = t2 * 2 + b
                    j = j0 + t
                    rb, sb = bufs[b]
                    rn, sn = bufs[1 - b]
                    nxt_row = jnp.where(t + 1 < _W, off + t + 1, offn)

                    @pl.when(j + 1 < nch)
                    def _():
                        pltpu.make_async_copy(
                            hs_hbm.at[idx_s.at[nxt_row]], rn, sn).start()

                    @pl.when(j < nch)
                    def _():
                        pltpu.make_async_copy(
                            hs_hbm.at[idx_s.at[off + t]], rb, sb).wait()
                        pltpu.sync_copy(rb, acc.at[idx_d.at[off + t]], add=True)

                return carry2

            lax.fori_loop(0, _W // 2, pair_body, 0)
            return carry

        lax.fori_loop(0, nwin, win_body, 0)
        plsc.subcore_barrier()
        pltpu.sync_copy(acc.at[pl.ds(r0, rpt)], out_hbm.at[c, pl.ds(r0, rpt)])

    return k(hs, src2d, dst2d)


def _tc_scale_matmul(x, w, degp):
    n, d = x.shape
    nw = degp.shape[0]
    blk = 1024

    def body(x_ref, w_ref, deg_ref, o_ref, dis_ref):
        deg = jnp.sum(deg_ref[...], axis=0) + 1.0
        disv = lax.rsqrt(deg).reshape(blk, 1)
        dis_ref[...] = disv
        h = jnp.dot(x_ref[...], w_ref[...], preferred_element_type=jnp.float32)
        o_ref[...] = h * disv

    return pl.pallas_call(
        body,
        grid=(n // blk,),
        in_specs=[
            pl.BlockSpec((blk, d), lambda i: (i, 0)),
            pl.BlockSpec((d, d), lambda i: (0, 0)),
            pl.BlockSpec((nw, blk), lambda i: (0, i)),
        ],
        out_specs=[
            pl.BlockSpec((blk, d), lambda i: (i, 0)),
            pl.BlockSpec((blk, 1), lambda i: (i, 0)),
        ],
        out_shape=[
            jax.ShapeDtypeStruct((n, d), jnp.float32),
            jax.ShapeDtypeStruct((n, 1), jnp.float32),
        ],
    )(x, w, degp)


def _tc_combine_selu_matmul(y, n, dis, b, w):
    d = y.shape[2]
    blk = 1000

    def body(y_ref, dis_ref, b_ref, w_ref, o_ref):
        t = dis_ref[...] * (y_ref[0] + y_ref[1]) + b_ref[...]
        a = _SELU_SCALE * jnp.where(t > 0, t, _SELU_ALPHA * (jnp.exp(t) - 1.0))
        h = jnp.dot(a, w_ref[...], preferred_element_type=jnp.float32)
        o_ref[...] = h * dis_ref[...]

    return pl.pallas_call(
        body,
        grid=(n // blk,),
        in_specs=[
            pl.BlockSpec((_NC, blk, d), lambda i: (0, i, 0)),
            pl.BlockSpec((blk, 1), lambda i: (i, 0)),
            pl.BlockSpec((1, d), lambda i: (0, 0)),
            pl.BlockSpec((d, d), lambda i: (0, 0)),
        ],
        out_specs=pl.BlockSpec((blk, d), lambda i: (i, 0)),
        out_shape=jax.ShapeDtypeStruct((n, d), jnp.float32),
    )(y, dis, b, w)


def _tc_combine(y, n, dis, b):
    d = y.shape[2]
    blk = 1000

    def body(y_ref, dis_ref, b_ref, o_ref):
        o_ref[...] = dis_ref[...] * (y_ref[0] + y_ref[1]) + b_ref[...]

    return pl.pallas_call(
        body,
        grid=(n // blk,),
        in_specs=[
            pl.BlockSpec((_NC, blk, d), lambda i: (0, i, 0)),
            pl.BlockSpec((blk, 1), lambda i: (i, 0)),
            pl.BlockSpec((1, d), lambda i: (0, 0)),
        ],
        out_specs=pl.BlockSpec((blk, d), lambda i: (i, 0)),
        out_shape=jax.ShapeDtypeStruct((n, d), jnp.float32),
    )(y, dis, b)


def kernel(x, adj_t, W1, b1, W2, b2):
    n, d = x.shape
    e = adj_t.shape[1]
    nchunks = e // _CHUNK
    maxc = -(-nchunks // _NW)
    maxc = -(-maxc // _W) * _W
    pad = _NW * maxc * _CHUNK - e

    src2d = jnp.concatenate(
        [adj_t[0], jnp.zeros((pad,), adj_t.dtype)]).reshape(_NW * maxc, _CHUNK)
    dst2d = jnp.concatenate(
        [adj_t[1], jnp.zeros((pad,), adj_t.dtype)]).reshape(_NW * maxc, _CHUNK)

    degp = _sc_degree(dst2d, nchunks)

    b1r = b1[None, :]
    b2r = b2[None, :]

    xp = jnp.pad(x, ((0, _NP - n), (0, 0)))
    hs1, dis = _tc_scale_matmul(xp, W1, degp)
    y1 = _sc_aggregate(hs1, src2d, dst2d, nchunks)
    hs2 = _tc_combine_selu_matmul(y1, n, dis, b1r, W2)
    y2 = _sc_aggregate(hs2, src2d, dst2d, nchunks)
    return _tc_combine(y2, n, dis, b2r)


# speedup vs baseline: 35.4922x; 1.0444x over previous
import functools

import jax
import jax.numpy as jnp
from jax import lax
from jax.experimental import pallas as pl
from jax.experimental.pallas import tpu as pltpu
from jax.experimental.pallas import tpu_sc as plsc

_NC = 2
_NS = 16
_NW = _NC * _NS
_LANES = 16
_CHUNK = 128
_NP = 10240

_SELU_SCALE = 1.0507009873554805
_SELU_ALPHA = 1.6732632423543772


def _sc_degree(adj2d, nchunks):
    maxc = adj2d.shape[1] // _NW
    mesh = plsc.VectorSubcoreMesh(core_axis_name="c", subcore_axis_name="s")

    @functools.partial(
        pl.kernel,
        mesh=mesh,
        out_type=jax.ShapeDtypeStruct((_NW, _NP), jnp.float32),
        scratch_types=[
            pltpu.VMEM((_NP,), jnp.float32),
            pltpu.VMEM((maxc, _CHUNK), jnp.int32),
        ],
        compiler_params=pltpu.CompilerParams(needs_layout_passes=False),
    )
    def k(adj_hbm, out_hbm, hist, idxb):
        c = lax.axis_index("c")
        s = lax.axis_index("s")
        wid = s * _NC + c
        c0 = pl.multiple_of(wid * maxc, 8)
        nch = jnp.minimum(jnp.maximum(nchunks - wid * maxc, 0), maxc)

        pltpu.sync_copy(adj_hbm.at[1, pl.ds(c0, maxc)], idxb)

        def zbody(i, carry):
            hist[pl.ds(i * _LANES, _LANES)] = jnp.zeros((_LANES,), jnp.float32)
            return carry

        lax.fori_loop(0, _NP // _LANES, zbody, 0)

        ones16 = jnp.full((_LANES,), 1.0, jnp.float32)

        def body(j, carry):
            @pl.when(j < nch)
            def _():
                for t in range(_CHUNK // _LANES):
                    idx16 = idxb[j, pl.ds(t * _LANES, _LANES)]
                    plsc.addupdate_scatter(hist, [idx16], ones16)

            return carry

        lax.fori_loop(0, maxc, body, 0)
        pltpu.sync_copy(hist, out_hbm.at[wid])

    return k(adj2d)


_W = 16


def _sc_aggregate(hs, adj2d, nchunks):
    n, d = hs.shape
    maxc = adj2d.shape[1] // _NW
    rpt = _NP // _NS
    rlast = n - (_NS - 1) * rpt
    mesh = plsc.VectorSubcoreMesh(core_axis_name="c", subcore_axis_name="s")

    @functools.partial(
        pl.kernel,
        mesh=mesh,
        out_type=jax.ShapeDtypeStruct((_NC, _NP, d), jnp.float32),
        scratch_types=[
            pltpu.VMEM_SHARED((_NP, d), jnp.float32),
            pltpu.VMEM((2 * _W, _CHUNK), jnp.int32),
            pltpu.VMEM((2 * _W, _CHUNK), jnp.int32),
            pltpu.VMEM((_CHUNK, d), jnp.float32),
            pltpu.VMEM((_CHUNK, d), jnp.float32),
            pltpu.SemaphoreType.DMA,
            pltpu.SemaphoreType.DMA,
            pltpu.SemaphoreType.DMA,
            pltpu.SemaphoreType.DMA,
        ],
    )
    def k(hs_hbm, adj_hbm, out_hbm,
          acc, idx_s, idx_d, rows0, rows1, sem0, sem1, semis, semid):
        c = lax.axis_index("c")
        s = lax.axis_index("s")
        wid = s * _NC + c
        c0 = pl.multiple_of(wid * maxc, 8)
        nch = jnp.minimum(jnp.maximum(nchunks - wid * maxc, 0), maxc)
        r0 = pl.multiple_of(s * rpt, 8)

        @pl.when(jnp.logical_and(c == 0, s < _NS - 1))
        def _():
            pltpu.sync_copy(hs_hbm.at[pl.ds(r0, rpt)], acc.at[pl.ds(r0, rpt)])

        @pl.when(jnp.logical_and(c == 0, s == _NS - 1))
        def _():
            pltpu.sync_copy(hs_hbm.at[pl.ds(r0, rlast)], acc.at[pl.ds(r0, rlast)])

        @pl.when(c == 1)
        def _():
            z16 = jnp.zeros((_LANES,), jnp.float32)

            def zb(r, carry):
                for t in range(d // _LANES):
                    rows0[r, pl.ds(t * _LANES, _LANES)] = z16
                return carry

            lax.fori_loop(0, _CHUNK, zb, 0)
            for p in range(rpt // _CHUNK):
                pltpu.sync_copy(
                    rows0, acc.at[pl.ds(pl.multiple_of(r0 + p * _CHUNK, 8),
                                        _CHUNK)])

        plsc.subcore_barrier()

        bufs = ((rows0, sem0), (rows1, sem1))
        nwin = maxc // _W

        pltpu.sync_copy(adj_hbm.at[0, pl.ds(c0, _W)], idx_s.at[pl.ds(0, _W)])
        pltpu.sync_copy(adj_hbm.at[1, pl.ds(c0, _W)], idx_d.at[pl.ds(0, _W)])

        @pl.when(0 < nch)
        def _():
            pltpu.make_async_copy(hs_hbm.at[idx_s.at[0]], rows0, sem0).start()

        def win_body(w, carry):
            par = w % 2
            off = par * _W
            offn = _W - off
            j0 = w * _W
            nb = pl.multiple_of(c0 + (w + 1) * _W, 8)

            @pl.when(w + 1 < nwin)
            def _():
                pltpu.make_async_copy(
                    adj_hbm.at[0, pl.ds(nb, _W)],
                    idx_s.at[pl.ds(offn, _W)], semis).start()
                pltpu.make_async_copy(
                    adj_hbm.at[1, pl.ds(nb, _W)],
                    idx_d.at[pl.ds(offn, _W)], semid).start()

            def pair_body(t2, carry2):
                @pl.when(jnp.logical_and(t2 == 3, w + 1 < nwin))
                def _():
                    pltpu.make_async_copy(
                        adj_hbm.at[0, pl.ds(nb, _W)],
                        idx_s.at[pl.ds(offn, _W)], semis).wait()
                    pltpu.make_async_copy(
                        adj_hbm.at[1, pl.ds(nb, _W)],
                        idx_d.at[pl.ds(offn, _W)], semid).wait()

                for b in range(2):
                    t = t2 * 2 + b
                    j = j0 + t
                    rb, sb = bufs[b]
                    rn, sn = bufs[1 - b]
                    nxt_row = jnp.where(t + 1 < _W, off + t + 1, offn)

                    @pl.when(j + 1 < nch)
                    def _():
                        pltpu.make_async_copy(
                            hs_hbm.at[idx_s.at[nxt_row]], rn, sn).start()

                    @pl.when(j < nch)
                    def _():
                        pltpu.make_async_copy(
                            hs_hbm.at[idx_s.at[off + t]], rb, sb).wait()
                        pltpu.sync_copy(rb, acc.at[idx_d.at[off + t]], add=True)

                return carry2

            lax.fori_loop(0, _W // 2, pair_body, 0)
            return carry

        lax.fori_loop(0, nwin, win_body, 0)
        plsc.subcore_barrier()
        pltpu.sync_copy(acc.at[pl.ds(r0, rpt)], out_hbm.at[c, pl.ds(r0, rpt)])

    return k(hs, adj2d)


def _tc_scale_matmul(x, w, degp):
    n, d = x.shape
    nw = degp.shape[0]
    blk = 1024

    def body(x_ref, w_ref, deg_ref, o_ref, dis_ref):
        deg = jnp.sum(deg_ref[...], axis=0) + 1.0
        disv = lax.rsqrt(deg).reshape(blk, 1)
        dis_ref[...] = disv
        h = jnp.dot(x_ref[...], w_ref[...], preferred_element_type=jnp.float32)
        o_ref[...] = h * disv

    return pl.pallas_call(
        body,
        grid=(-(-n // blk),),
        in_specs=[
            pl.BlockSpec((blk, d), lambda i: (i, 0)),
            pl.BlockSpec((d, d), lambda i: (0, 0)),
            pl.BlockSpec((nw, blk), lambda i: (0, i)),
        ],
        out_specs=[
            pl.BlockSpec((blk, d), lambda i: (i, 0)),
            pl.BlockSpec((blk, 1), lambda i: (i, 0)),
        ],
        out_shape=[
            jax.ShapeDtypeStruct((n, d), jnp.float32),
            jax.ShapeDtypeStruct((n, 1), jnp.float32),
        ],
    )(x, w, degp)


def _tc_combine_selu_matmul(y, n, dis, b, w):
    d = y.shape[2]
    blk = 1000

    def body(y_ref, dis_ref, b_ref, w_ref, o_ref):
        t = dis_ref[...] * (y_ref[0] + y_ref[1]) + b_ref[...]
        a = _SELU_SCALE * jnp.where(t > 0, t, _SELU_ALPHA * (jnp.exp(t) - 1.0))
        h = jnp.dot(a, w_ref[...], preferred_element_type=jnp.float32)
        o_ref[...] = h * dis_ref[...]

    return pl.pallas_call(
        body,
        grid=(n // blk,),
        in_specs=[
            pl.BlockSpec((_NC, blk, d), lambda i: (0, i, 0)),
            pl.BlockSpec((blk, 1), lambda i: (i, 0)),
            pl.BlockSpec((d,), lambda i: (0,)),
            pl.BlockSpec((d, d), lambda i: (0, 0)),
        ],
        out_specs=pl.BlockSpec((blk, d), lambda i: (i, 0)),
        out_shape=jax.ShapeDtypeStruct((n, d), jnp.float32),
    )(y, dis, b, w)


def _tc_combine(y, n, dis, b):
    d = y.shape[2]
    blk = 1000

    def body(y_ref, dis_ref, b_ref, o_ref):
        o_ref[...] = dis_ref[...] * (y_ref[0] + y_ref[1]) + b_ref[...]

    return pl.pallas_call(
        body,
        grid=(n // blk,),
        in_specs=[
            pl.BlockSpec((_NC, blk, d), lambda i: (0, i, 0)),
            pl.BlockSpec((blk, 1), lambda i: (i, 0)),
            pl.BlockSpec((d,), lambda i: (0,)),
        ],
        out_specs=pl.BlockSpec((blk, d), lambda i: (i, 0)),
        out_shape=jax.ShapeDtypeStruct((n, d), jnp.float32),
    )(y, dis, b)


def kernel(x, adj_t, W1, b1, W2, b2):
    n, d = x.shape
    e = adj_t.shape[1]
    nchunks = e // _CHUNK
    maxc = -(-nchunks // _NW)
    maxc = -(-maxc // _W) * _W
    pad = _NW * maxc * _CHUNK - e

    adj2d = jnp.pad(adj_t.reshape(2, nchunks, _CHUNK),
                    ((0, 0), (0, _NW * maxc - nchunks), (0, 0)))
    degp = _sc_degree(adj2d, nchunks)

    hs1, dis = _tc_scale_matmul(x, W1, degp)
    y1 = _sc_aggregate(hs1, adj2d, nchunks)
    hs2 = _tc_combine_selu_matmul(y1, n, dis, b1, W2)
    y2 = _sc_aggregate(hs2, adj2d, nchunks)
    return _tc_combine(y2, n, dis, b2)


# speedup vs baseline: 36.3377x; 1.0238x over previous
import functools

import jax
import jax.numpy as jnp
from jax import lax
from jax.experimental import pallas as pl
from jax.experimental.pallas import tpu as pltpu
from jax.experimental.pallas import tpu_sc as plsc

_NC = 2
_NS = 16
_NW = _NC * _NS
_LANES = 16
_CHUNK = 128
_NP = 10240

_SELU_SCALE = 1.0507009873554805
_SELU_ALPHA = 1.6732632423543772


def _sc_degree(adj2d, nchunks):
    maxc = adj2d.shape[1] // _NW
    mesh = plsc.VectorSubcoreMesh(core_axis_name="c", subcore_axis_name="s")

    @functools.partial(
        pl.kernel,
        mesh=mesh,
        out_type=jax.ShapeDtypeStruct((_NW, _NP), jnp.float32),
        scratch_types=[
            pltpu.VMEM((_NP,), jnp.float32),
            pltpu.VMEM((maxc, _CHUNK), jnp.int32),
        ],
        compiler_params=pltpu.CompilerParams(needs_layout_passes=False),
    )
    def k(adj_hbm, out_hbm, hist, idxb):
        c = lax.axis_index("c")
        s = lax.axis_index("s")
        wid = s * _NC + c
        c0 = pl.multiple_of(wid * maxc, 8)
        nch = jnp.minimum(jnp.maximum(nchunks - wid * maxc, 0), maxc)

        pltpu.sync_copy(adj_hbm.at[1, pl.ds(c0, maxc)], idxb)

        def zbody(i, carry):
            hist[pl.ds(i * _LANES, _LANES)] = jnp.zeros((_LANES,), jnp.float32)
            return carry

        lax.fori_loop(0, _NP // _LANES, zbody, 0)

        ones16 = jnp.full((_LANES,), 1.0, jnp.float32)

        def body(j, carry):
            @pl.when(j < nch)
            def _():
                for t in range(_CHUNK // _LANES):
                    idx16 = idxb[j, pl.ds(t * _LANES, _LANES)]
                    plsc.addupdate_scatter(hist, [idx16], ones16)

            return carry

        lax.fori_loop(0, maxc, body, 0)
        pltpu.sync_copy(hist, out_hbm.at[wid])

    return k(adj2d)


_W = 16


def _sc_aggregate(hs, adj2d, nchunks):
    n, d = hs.shape
    maxc = adj2d.shape[1] // _NW
    rpt = _NP // _NS
    rlast = n - (_NS - 1) * rpt
    mesh = plsc.VectorSubcoreMesh(core_axis_name="c", subcore_axis_name="s")

    @functools.partial(
        pl.kernel,
        mesh=mesh,
        out_type=jax.ShapeDtypeStruct((_NC, _NP, d), jnp.float32),
        scratch_types=[
            pltpu.VMEM_SHARED((_NP, d), jnp.float32),
            pltpu.VMEM((2 * _W, _CHUNK), jnp.int32),
            pltpu.VMEM((2 * _W, _CHUNK), jnp.int32),
            pltpu.VMEM((_CHUNK, d), jnp.float32),
            pltpu.VMEM((_CHUNK, d), jnp.float32),
            pltpu.SemaphoreType.DMA,
            pltpu.SemaphoreType.DMA,
            pltpu.SemaphoreType.DMA,
            pltpu.SemaphoreType.DMA,
        ],
    )
    def k(hs_hbm, adj_hbm, out_hbm,
          acc, idx_s, idx_d, rows0, rows1, sem0, sem1, semis, semid):
        c = lax.axis_index("c")
        s = lax.axis_index("s")
        wid = s * _NC + c
        c0 = pl.multiple_of(wid * maxc, 8)
        nch = jnp.minimum(jnp.maximum(nchunks - wid * maxc, 0), maxc)
        r0 = pl.multiple_of(s * rpt, 8)

        @pl.when(jnp.logical_and(c == 0, s < _NS - 1))
        def _():
            pltpu.sync_copy(hs_hbm.at[pl.ds(r0, rpt)], acc.at[pl.ds(r0, rpt)])

        @pl.when(jnp.logical_and(c == 0, s == _NS - 1))
        def _():
            pltpu.sync_copy(hs_hbm.at[pl.ds(r0, rlast)], acc.at[pl.ds(r0, rlast)])

        @pl.when(c == 1)
        def _():
            z16 = jnp.zeros((_LANES,), jnp.float32)

            def zb(r, carry):
                for t in range(d // _LANES):
                    rows0[r, pl.ds(t * _LANES, _LANES)] = z16
                return carry

            lax.fori_loop(0, _CHUNK, zb, 0)
            for p in range(rpt // _CHUNK):
                pltpu.sync_copy(
                    rows0, acc.at[pl.ds(pl.multiple_of(r0 + p * _CHUNK, 8),
                                        _CHUNK)])

        plsc.subcore_barrier()

        bufs = ((rows0, sem0), (rows1, sem1))
        nwin = maxc // _W

        pltpu.sync_copy(adj_hbm.at[0, pl.ds(c0, _W)], idx_s.at[pl.ds(0, _W)])
        pltpu.sync_copy(adj_hbm.at[1, pl.ds(c0, _W)], idx_d.at[pl.ds(0, _W)])

        @pl.when(0 < nch)
        def _():
            pltpu.make_async_copy(hs_hbm.at[idx_s.at[0]], rows0, sem0).start()

        def win_body(w, carry):
            par = w % 2
            off = par * _W
            offn = _W - off
            j0 = w * _W
            nb = pl.multiple_of(c0 + (w + 1) * _W, 8)

            @pl.when(w + 1 < nwin)
            def _():
                pltpu.make_async_copy(
                    adj_hbm.at[0, pl.ds(nb, _W)],
                    idx_s.at[pl.ds(offn, _W)], semis).start()
                pltpu.make_async_copy(
                    adj_hbm.at[1, pl.ds(nb, _W)],
                    idx_d.at[pl.ds(offn, _W)], semid).start()

            def pair_body(t2, carry2):
                @pl.when(jnp.logical_and(t2 == 3, w + 1 < nwin))
                def _():
                    pltpu.make_async_copy(
                        adj_hbm.at[0, pl.ds(nb, _W)],
                        idx_s.at[pl.ds(offn, _W)], semis).wait()
                    pltpu.make_async_copy(
                        adj_hbm.at[1, pl.ds(nb, _W)],
                        idx_d.at[pl.ds(offn, _W)], semid).wait()

                for b in range(2):
                    t = t2 * 2 + b
                    j = j0 + t
                    rb, sb = bufs[b]
                    rn, sn = bufs[1 - b]
                    nxt_row = jnp.where(t + 1 < _W, off + t + 1, offn)

                    @pl.when(j + 1 < nch)
                    def _():
                        pltpu.make_async_copy(
                            hs_hbm.at[idx_s.at[nxt_row]], rn, sn).start()

                    @pl.when(j < nch)
                    def _():
                        pltpu.make_async_copy(
                            hs_hbm.at[idx_s.at[off + t]], rb, sb).wait()
                        pltpu.sync_copy(rb, acc.at[idx_d.at[off + t]], add=True)

                return carry2

            lax.fori_loop(0, _W // 2, pair_body, 0)
            return carry

        lax.fori_loop(0, nwin, win_body, 0)
        plsc.subcore_barrier()
        pltpu.sync_copy(acc.at[pl.ds(r0, rpt)], out_hbm.at[c, pl.ds(r0, rpt)])

    return k(hs, adj2d)


def _tc_scale_matmul(x, w, degp):
    n, d = x.shape
    nw = degp.shape[0]
    blk = 2048

    def body(x_ref, w_ref, deg_ref, o_ref, dis_ref):
        deg = jnp.sum(deg_ref[...], axis=0) + 1.0
        disv = lax.rsqrt(deg).reshape(blk, 1)
        dis_ref[...] = disv
        h = jnp.dot(x_ref[...], w_ref[...], preferred_element_type=jnp.float32)
        o_ref[...] = h * disv

    return pl.pallas_call(
        body,
        grid=(-(-n // blk),),
        in_specs=[
            pl.BlockSpec((blk, d), lambda i: (i, 0)),
            pl.BlockSpec((d, d), lambda i: (0, 0)),
            pl.BlockSpec((nw, blk), lambda i: (0, i)),
        ],
        out_specs=[
            pl.BlockSpec((blk, d), lambda i: (i, 0)),
            pl.BlockSpec((blk, 1), lambda i: (i, 0)),
        ],
        out_shape=[
            jax.ShapeDtypeStruct((n, d), jnp.float32),
            jax.ShapeDtypeStruct((n, 1), jnp.float32),
        ],
    )(x, w, degp)


def _tc_combine_selu_matmul(y, n, dis, b, w):
    d = y.shape[2]
    blk = 2000

    def body(y_ref, dis_ref, b_ref, w_ref, o_ref):
        t = dis_ref[...] * (y_ref[0] + y_ref[1]) + b_ref[...]
        a = _SELU_SCALE * jnp.where(t > 0, t, _SELU_ALPHA * (jnp.exp(t) - 1.0))
        h = jnp.dot(a, w_ref[...], preferred_element_type=jnp.float32)
        o_ref[...] = h * dis_ref[...]

    return pl.pallas_call(
        body,
        grid=(n // blk,),
        in_specs=[
            pl.BlockSpec((_NC, blk, d), lambda i: (0, i, 0)),
            pl.BlockSpec((blk, 1), lambda i: (i, 0)),
            pl.BlockSpec((d,), lambda i: (0,)),
            pl.BlockSpec((d, d), lambda i: (0, 0)),
        ],
        out_specs=pl.BlockSpec((blk, d), lambda i: (i, 0)),
        out_shape=jax.ShapeDtypeStruct((n, d), jnp.float32),
    )(y, dis, b, w)


def _tc_combine(y, n, dis, b):
    d = y.shape[2]
    blk = 2000

    def body(y_ref, dis_ref, b_ref, o_ref):
        o_ref[...] = dis_ref[...] * (y_ref[0] + y_ref[1]) + b_ref[...]

    return pl.pallas_call(
        body,
        grid=(n // blk,),
        in_specs=[
            pl.BlockSpec((_NC, blk, d), lambda i: (0, i, 0)),
            pl.BlockSpec((blk, 1), lambda i: (i, 0)),
            pl.BlockSpec((d,), lambda i: (0,)),
        ],
        out_specs=pl.BlockSpec((blk, d), lambda i: (i, 0)),
        out_shape=jax.ShapeDtypeStruct((n, d), jnp.float32),
    )(y, dis, b)


def kernel(x, adj_t, W1, b1, W2, b2):
    n, d = x.shape
    e = adj_t.shape[1]
    nchunks = e // _CHUNK
    maxc = -(-nchunks // _NW)
    maxc = -(-maxc // _W) * _W
    pad = _NW * maxc * _CHUNK - e

    adj2d = jnp.pad(adj_t.reshape(2, nchunks, _CHUNK),
                    ((0, 0), (0, _NW * maxc - nchunks), (0, 0)))
    degp = _sc_degree(adj2d, nchunks)

    hs1, dis = _tc_scale_matmul(x, W1, degp)
    y1 = _sc_aggregate(hs1, adj2d, nchunks)
    hs2 = _tc_combine_selu_matmul(y1, n, dis, b1, W2)
    y2 = _sc_aggregate(hs2, adj2d, nchunks)
    return _tc_combine(y2, n, dis, b2)


# speedup vs baseline: 36.6986x; 1.0099x over previous
import functools

import jax
import jax.numpy as jnp
from jax import lax
from jax.experimental import pallas as pl
from jax.experimental.pallas import tpu as pltpu
from jax.experimental.pallas import tpu_sc as plsc

_NC = 2
_NS = 16
_NW = _NC * _NS
_LANES = 16
_CHUNK = 128
_NP = 10240

_SELU_SCALE = 1.0507009873554805
_SELU_ALPHA = 1.6732632423543772


def _sc_degree(adj2d, nchunks):
    maxc = adj2d.shape[1] // _NW
    mesh = plsc.VectorSubcoreMesh(core_axis_name="c", subcore_axis_name="s")

    @functools.partial(
        pl.kernel,
        mesh=mesh,
        out_type=jax.ShapeDtypeStruct((_NW, _NP), jnp.float32),
        scratch_types=[
            pltpu.VMEM((_NP,), jnp.float32),
            pltpu.VMEM((maxc, _CHUNK), jnp.int32),
        ],
        compiler_params=pltpu.CompilerParams(needs_layout_passes=False),
    )
    def k(adj_hbm, out_hbm, hist, idxb):
        c = lax.axis_index("c")
        s = lax.axis_index("s")
        wid = s * _NC + c
        c0 = pl.multiple_of(wid * maxc, 8)
        nch = jnp.minimum(jnp.maximum(nchunks - wid * maxc, 0), maxc)

        pltpu.sync_copy(adj_hbm.at[1, pl.ds(c0, maxc)], idxb)

        z16 = jnp.zeros((_LANES,), jnp.float32)

        def zbody(i, carry):
            base = i * 8 * _LANES
            for t in range(8):
                hist[pl.ds(base + t * _LANES, _LANES)] = z16
            return carry

        lax.fori_loop(0, _NP // (8 * _LANES), zbody, 0)

        ones16 = jnp.full((_LANES,), 1.0, jnp.float32)

        def body(j, carry):
            @pl.when(j < nch)
            def _():
                for t in range(_CHUNK // _LANES):
                    idx16 = idxb[j, pl.ds(t * _LANES, _LANES)]
                    plsc.addupdate_scatter(hist, [idx16], ones16)

            return carry

        lax.fori_loop(0, maxc, body, 0)
        pltpu.sync_copy(hist, out_hbm.at[wid])

    return k(adj2d)


_W = 16


def _sc_aggregate(hs, adj2d, nchunks):
    n, d = hs.shape
    maxc = adj2d.shape[1] // _NW
    rpt = _NP // _NS
    rlast = n - (_NS - 1) * rpt
    mesh = plsc.VectorSubcoreMesh(core_axis_name="c", subcore_axis_name="s")

    @functools.partial(
        pl.kernel,
        mesh=mesh,
        out_type=jax.ShapeDtypeStruct((_NC, _NP, d), jnp.float32),
        scratch_types=[
            pltpu.VMEM_SHARED((_NP, d), jnp.float32),
            pltpu.VMEM((2 * _W, _CHUNK), jnp.int32),
            pltpu.VMEM((2 * _W, _CHUNK), jnp.int32),
            pltpu.VMEM((_CHUNK, d), jnp.float32),
            pltpu.VMEM((_CHUNK, d), jnp.float32),
            pltpu.SemaphoreType.DMA,
            pltpu.SemaphoreType.DMA,
            pltpu.SemaphoreType.DMA,
            pltpu.SemaphoreType.DMA,
        ],
    )
    def k(hs_hbm, adj_hbm, out_hbm,
          acc, idx_s, idx_d, rows0, rows1, sem0, sem1, semis, semid):
        c = lax.axis_index("c")
        s = lax.axis_index("s")
        wid = s * _NC + c
        c0 = pl.multiple_of(wid * maxc, 8)
        nch = jnp.minimum(jnp.maximum(nchunks - wid * maxc, 0), maxc)
        r0 = pl.multiple_of(s * rpt, 8)

        @pl.when(jnp.logical_and(c == 0, s < _NS - 1))
        def _():
            pltpu.sync_copy(hs_hbm.at[pl.ds(r0, rpt)], acc.at[pl.ds(r0, rpt)])

        @pl.when(jnp.logical_and(c == 0, s == _NS - 1))
        def _():
            pltpu.sync_copy(hs_hbm.at[pl.ds(r0, rlast)], acc.at[pl.ds(r0, rlast)])

        @pl.when(c == 1)
        def _():
            z16 = jnp.zeros((_LANES,), jnp.float32)

            def zb(r, carry):
                for t in range(d // _LANES):
                    rows0[r, pl.ds(t * _LANES, _LANES)] = z16
                return carry

            lax.fori_loop(0, _CHUNK, zb, 0)
            for p in range(rpt // _CHUNK):
                pltpu.sync_copy(
                    rows0, acc.at[pl.ds(pl.multiple_of(r0 + p * _CHUNK, 8),
                                        _CHUNK)])

        plsc.subcore_barrier()

        bufs = ((rows0, sem0), (rows1, sem1))
        nwin = maxc // _W

        pltpu.sync_copy(adj_hbm.at[0, pl.ds(c0, _W)], idx_s.at[pl.ds(0, _W)])
        pltpu.sync_copy(adj_hbm.at[1, pl.ds(c0, _W)], idx_d.at[pl.ds(0, _W)])

        @pl.when(0 < nch)
        def _():
            pltpu.make_async_copy(hs_hbm.at[idx_s.at[0]], rows0, sem0).start()

        def win_body(w, carry):
            par = w % 2
            off = par * _W
            offn = _W - off
            j0 = w * _W
            nb = pl.multiple_of(c0 + (w + 1) * _W, 8)

            @pl.when(w + 1 < nwin)
            def _():
                pltpu.make_async_copy(
                    adj_hbm.at[0, pl.ds(nb, _W)],
                    idx_s.at[pl.ds(offn, _W)], semis).start()
                pltpu.make_async_copy(
                    adj_hbm.at[1, pl.ds(nb, _W)],
                    idx_d.at[pl.ds(offn, _W)], semid).start()

            def pair_body(t2, carry2):
                @pl.when(jnp.logical_and(t2 == 3, w + 1 < nwin))
                def _():
                    pltpu.make_async_copy(
                        adj_hbm.at[0, pl.ds(nb, _W)],
                        idx_s.at[pl.ds(offn, _W)], semis).wait()
                    pltpu.make_async_copy(
                        adj_hbm.at[1, pl.ds(nb, _W)],
                        idx_d.at[pl.ds(offn, _W)], semid).wait()

                for b in range(2):
                    t = t2 * 2 + b
                    j = j0 + t
                    rb, sb = bufs[b]
                    rn, sn = bufs[1 - b]
                    nxt_row = jnp.where(t + 1 < _W, off + t + 1, offn)

                    @pl.when(j + 1 < nch)
                    def _():
                        pltpu.make_async_copy(
                            hs_hbm.at[idx_s.at[nxt_row]], rn, sn).start()

                    @pl.when(j < nch)
                    def _():
                        pltpu.make_async_copy(
                            hs_hbm.at[idx_s.at[off + t]], rb, sb).wait()
                        pltpu.sync_copy(rb, acc.at[idx_d.at[off + t]], add=True)

                return carry2

            lax.fori_loop(0, _W // 2, pair_body, 0)
            return carry

        lax.fori_loop(0, nwin, win_body, 0)
        plsc.subcore_barrier()
        pltpu.sync_copy(acc.at[pl.ds(r0, rpt)], out_hbm.at[c, pl.ds(r0, rpt)])

    return k(hs, adj2d)


def _tc_scale_matmul(x, w, degp):
    n, d = x.shape
    nw = degp.shape[0]
    blk = 2048

    def body(x_ref, w_ref, deg_ref, o_ref, dis_ref):
        deg = jnp.sum(deg_ref[...], axis=0) + 1.0
        disv = lax.rsqrt(deg).reshape(blk, 1)
        dis_ref[...] = disv
        h = jnp.dot(x_ref[...], w_ref[...], preferred_element_type=jnp.float32)
        o_ref[...] = h * disv

    return pl.pallas_call(
        body,
        grid=(-(-n // blk),),
        in_specs=[
            pl.BlockSpec((blk, d), lambda i: (i, 0)),
            pl.BlockSpec((d, d), lambda i: (0, 0)),
            pl.BlockSpec((nw, blk), lambda i: (0, i)),
        ],
        out_specs=[
            pl.BlockSpec((blk, d), lambda i: (i, 0)),
            pl.BlockSpec((blk, 1), lambda i: (i, 0)),
        ],
        out_shape=[
            jax.ShapeDtypeStruct((n, d), jnp.float32),
            jax.ShapeDtypeStruct((n, 1), jnp.float32),
        ],
    )(x, w, degp)


def _tc_combine_selu_matmul(y, n, dis, b, w):
    d = y.shape[2]
    blk = 2000

    def body(y_ref, dis_ref, b_ref, w_ref, o_ref):
        t = dis_ref[...] * (y_ref[0] + y_ref[1]) + b_ref[...]
        a = _SELU_SCALE * jnp.where(t > 0, t, _SELU_ALPHA * (jnp.exp(t) - 1.0))
        h = jnp.dot(a, w_ref[...], preferred_element_type=jnp.float32)
        o_ref[...] = h * dis_ref[...]

    return pl.pallas_call(
        body,
        grid=(n // blk,),
        in_specs=[
            pl.BlockSpec((_NC, blk, d), lambda i: (0, i, 0)),
            pl.BlockSpec((blk, 1), lambda i: (i, 0)),
            pl.BlockSpec((d,), lambda i: (0,)),
            pl.BlockSpec((d, d), lambda i: (0, 0)),
        ],
        out_specs=pl.BlockSpec((blk, d), lambda i: (i, 0)),
        out_shape=jax.ShapeDtypeStruct((n, d), jnp.float32),
    )(y, dis, b, w)


def _tc_combine(y, n, dis, b):
    d = y.shape[2]
    blk = 2000

    def body(y_ref, dis_ref, b_ref, o_ref):
        o_ref[...] = dis_ref[...] * (y_ref[0] + y_ref[1]) + b_ref[...]

    return pl.pallas_call(
        body,
        grid=(n // blk,),
        in_specs=[
            pl.BlockSpec((_NC, blk, d), lambda i: (0, i, 0)),
            pl.BlockSpec((blk, 1), lambda i: (i, 0)),
            pl.BlockSpec((d,), lambda i: (0,)),
        ],
        out_specs=pl.BlockSpec((blk, d), lambda i: (i, 0)),
        out_shape=jax.ShapeDtypeStruct((n, d), jnp.float32),
    )(y, dis, b)


def kernel(x, adj_t, W1, b1, W2, b2):
    n, d = x.shape
    e = adj_t.shape[1]
    nchunks = e // _CHUNK
    maxc = -(-nchunks // _NW)
    maxc = -(-maxc // _W) * _W
    pad = _NW * maxc * _CHUNK - e

    adj2d = jnp.pad(adj_t.reshape(2, nchunks, _CHUNK),
                    ((0, 0), (0, _NW * maxc - nchunks), (0, 0)))
    degp = _sc_degree(adj2d, nchunks)

    hs1, dis = _tc_scale_matmul(x, W1, degp)
    y1 = _sc_aggregate(hs1, adj2d, nchunks)
    hs2 = _tc_combine_selu_matmul(y1, n, dis, b1, W2)
    y2 = _sc_aggregate(hs2, adj2d, nchunks)
    return _tc_combine(y2, n, dis, b2)
